# baseline (speedup 1.0000x reference)
import functools

import jax
import jax.numpy as jnp
import numpy as np
from jax import lax
from jax.experimental import pallas as pl
from jax.experimental.pallas import tpu as pltpu
from jax.experimental.pallas import tpu_sc as plsc


def _t2_matmul(table, wcat, bias2, V, D, F):
    def body(a_ref, b_ref, bias_ref, o_ref):
        acc = jnp.dot(a_ref[...], b_ref[...], preferred_element_type=jnp.float32)
        sel = (pl.program_id(0) == 0).astype(jnp.float32)
        o_ref[...] = (acc + bias_ref[...] * sel).astype(jnp.bfloat16)

    return pl.pallas_call(
        body,
        grid=(F,),
        in_specs=[
            pl.BlockSpec((V, D), lambda k: (0, 0)),
            pl.BlockSpec((D, D), lambda k: (0, k)),
            pl.BlockSpec((1, D), lambda k: (0, 0)),
        ],
        out_specs=pl.BlockSpec((V, D), lambda k: (k, 0)),
        out_shape=jax.ShapeDtypeStruct((F * V, D), jnp.bfloat16),
    )(table, wcat, bias2)


def _sc_gather_sum(t2r, ids3, R, V, D, F, NC, NS):
    NW = NC * NS
    rows_w = R // NW
    CH = 8
    CHUNKS = rows_w // CH
    GR = CH * F
    DW = D // 2
    mesh = plsc.VectorSubcoreMesh(core_axis_name="c", subcore_axis_name="s")

    @functools.partial(
        pl.kernel,
        mesh=mesh,
        out_type=jax.ShapeDtypeStruct((R, D), jnp.float32),
        scratch_types=[
            pltpu.VMEM((CHUNKS, GR), jnp.int32),
            pltpu.VMEM((2, GR, DW), jnp.int32),
            pltpu.VMEM((2, CH, D), jnp.float32),
            pltpu.SemaphoreType.DMA,
            pltpu.SemaphoreType.DMA,
            pltpu.SemaphoreType.DMA,
            pltpu.SemaphoreType.DMA,
        ],
        compiler_params=pltpu.CompilerParams(needs_layout_passes=False),
    )
    def k(t2_hbm, ids_hbm, out_hbm, idx_v, rows_v, acc_v, g0, g1, o0, o1):
        wid = lax.axis_index("s") * NC + lax.axis_index("c")
        pltpu.sync_copy(ids_hbm.at[wid], idx_v)
        pat = lax.rem(lax.iota(jnp.int32, 16), jnp.full((16,), F, jnp.int32))
        patv = pat * V
        for c in range(CHUNKS):
            for i in range(GR // 16):
                sl = pl.ds(i * 16, 16)
                idx_v[c, sl] = idx_v[c, sl] + patv

        gsems = (g0, g1)
        osems = (o0, o1)
        base = wid * rows_w
        shamt = jnp.full((16,), 16, jnp.int32)
        hmask = jnp.full((16,), -65536, jnp.int32)

        def gather_desc(c, par):
            return pltpu.make_async_copy(
                t2_hbm.at[idx_v.at[c]], rows_v.at[par], gsems[par]
            )

        def out_desc(c, par):
            return pltpu.make_async_copy(
                acc_v.at[par], out_hbm.at[pl.ds(base + c * CH, CH)], osems[par]
            )

        def halves(x):
            lo = plsc.bitcast(lax.shift_left(x, shamt), jnp.float32)
            hi = plsc.bitcast(jnp.bitwise_and(x, hmask), jnp.float32)
            return lo, hi

        NG = D // 32
        gather_desc(0, 0).start()

        def pairbody(h, carry):
            for par in (0, 1):
                c = 2 * h + par
                gather_desc(c, par).wait()

                @pl.when(c + 1 < CHUNKS)
                def _():
                    gather_desc(c + 1, 1 - par).start()

                @pl.when(c >= 2)
                def _():
                    out_desc(c - 2, par).wait()

                rb = rows_v.at[par]
                ab = acc_v.at[par]

                def accrow(j, cc):
                    row = F * j
                    for g in range(NG):
                        sl = pl.ds(g * 16, 16)
                        l0, h0 = halves(rb[row, sl])
                        l1, h1 = halves(rb[row + 1, sl])
                        l2, h2 = halves(rb[row + 2, sl])
                        l3, h3 = halves(rb[row + 3, sl])
                        ab[j, pl.ds(g * 32, 16)] = (l0 + l1) + (l2 + l3)
                        ab[j, pl.ds(g * 32 + 16, 16)] = (h0 + h1) + (h2 + h3)
                    return cc

                lax.fori_loop(0, CH, accrow, 0)
                out_desc(c, par).start()
            return carry

        lax.fori_loop(0, CHUNKS // 2, pairbody, 0)
        out_desc(CHUNKS - 2, 0).wait()
        out_desc(CHUNKS - 1, 1).wait()

    return k(t2r, ids3)


def _interleave_perm(D):
    pcol = np.empty(D, np.int32)
    i = np.arange(16)
    for G in range(D // 32):
        pcol[32 * G + 2 * i] = 32 * G + i
        pcol[32 * G + 2 * i + 1] = 32 * G + 16 + i
    return pcol


def kernel(table, conv_w, conv_b, input_ids):
    V, D = table.shape
    F = conv_w.shape[2]
    B, S = input_ids.shape
    R = B * S // F

    info = plsc.get_sparse_core_info()
    NC, NS = info.num_cores, info.num_subcores
    NW = NC * NS

    pcol = jnp.asarray(_interleave_perm(D))
    wcatm = conv_w[pcol].transpose(1, 2, 0).reshape(D, F * D)
    bias2 = conv_b[pcol].reshape(1, D)
    t2 = _t2_matmul(table, wcatm, bias2, V, D, F)
    t2r = lax.bitcast_convert_type(t2.reshape(F * V, D // 2, 2), jnp.int32)

    rows_w = R // NW
    CH = 8
    ids3 = input_ids.reshape(NW, rows_w // CH, CH * F).astype(jnp.int32)
    out_flat = _sc_gather_sum(t2r, ids3, R, V, D, F, NC, NS)
    return out_flat.reshape(B, S // F, D)


# speedup vs baseline: 2.8334x; 2.8334x over previous
import functools

import jax
import jax.numpy as jnp
import numpy as np
from jax import lax
from jax.experimental import pallas as pl
from jax.experimental.pallas import tpu as pltpu
from jax.experimental.pallas import tpu_sc as plsc


def _t2_matmul(table, wcat, bias2, V, D, F):
    H = D // 2

    def body(a_ref, b_ref, bias_ref, o_ref):
        acc = jnp.dot(a_ref[...], b_ref[...], preferred_element_type=jnp.float32)
        sel = (pl.program_id(0) == 0).astype(jnp.float32)
        acc = acc + bias_ref[...] * sel
        lo = jax.lax.bitcast_convert_type(
            acc[:, :H].astype(jnp.bfloat16), jnp.uint16
        ).astype(jnp.uint32)
        hi = jax.lax.bitcast_convert_type(
            acc[:, H:].astype(jnp.bfloat16), jnp.uint16
        ).astype(jnp.uint32)
        o_ref[...] = jax.lax.bitcast_convert_type(lo | (hi << 16), jnp.int32)

    return pl.pallas_call(
        body,
        grid=(F,),
        in_specs=[
            pl.BlockSpec((V, D), lambda k: (0, 0)),
            pl.BlockSpec((D, D), lambda k: (0, k)),
            pl.BlockSpec((1, D), lambda k: (0, 0)),
        ],
        out_specs=pl.BlockSpec((V, H), lambda k: (k, 0)),
        out_shape=jax.ShapeDtypeStruct((F * V, H), jnp.int32),
    )(table, wcat, bias2)


def _sc_gather_sum(t2r, ids3, R, V, D, F, NC, NS):
    NW = NC * NS
    rows_w = R // NW
    CH = 8
    CHUNKS = rows_w // CH
    GR = CH * F
    DW = D // 2
    mesh = plsc.VectorSubcoreMesh(core_axis_name="c", subcore_axis_name="s")

    @functools.partial(
        pl.kernel,
        mesh=mesh,
        out_type=jax.ShapeDtypeStruct((R, D), jnp.float32),
        scratch_types=[
            pltpu.VMEM((CHUNKS, GR), jnp.int32),
            pltpu.VMEM((2, GR, DW), jnp.int32),
            pltpu.VMEM((2, CH, D), jnp.float32),
            pltpu.SemaphoreType.DMA,
            pltpu.SemaphoreType.DMA,
            pltpu.SemaphoreType.DMA,
            pltpu.SemaphoreType.DMA,
        ],
        compiler_params=pltpu.CompilerParams(needs_layout_passes=False),
    )
    def k(t2_hbm, ids_hbm, out_hbm, idx_v, rows_v, acc_v, g0, g1, o0, o1):
        wid = lax.axis_index("s") * NC + lax.axis_index("c")
        pltpu.sync_copy(ids_hbm.at[wid], idx_v)
        pat = lax.rem(lax.iota(jnp.int32, 16), jnp.full((16,), F, jnp.int32))
        patv = pat * V
        for c in range(CHUNKS):
            for i in range(GR // 16):
                sl = pl.ds(i * 16, 16)
                idx_v[c, sl] = idx_v[c, sl] + patv

        gsems = (g0, g1)
        osems = (o0, o1)
        base = wid * rows_w
        shamt = jnp.full((16,), 16, jnp.int32)
        hmask = jnp.full((16,), -65536, jnp.int32)

        def gather_desc(c, par):
            return pltpu.make_async_copy(
                t2_hbm.at[idx_v.at[c]], rows_v.at[par], gsems[par]
            )

        def out_desc(c, par):
            return pltpu.make_async_copy(
                acc_v.at[par], out_hbm.at[pl.ds(base + c * CH, CH)], osems[par]
            )

        def halves(x):
            lo = plsc.bitcast(lax.shift_left(x, shamt), jnp.float32)
            hi = plsc.bitcast(jnp.bitwise_and(x, hmask), jnp.float32)
            return lo, hi

        NG = DW // 16
        gather_desc(0, 0).start()

        def pairbody(h, carry):
            for par in (0, 1):
                c = 2 * h + par
                gather_desc(c, par).wait()

                @pl.when(c + 1 < CHUNKS)
                def _():
                    gather_desc(c + 1, 1 - par).start()

                @pl.when(c >= 2)
                def _():
                    out_desc(c - 2, par).wait()

                rb = rows_v.at[par]
                ab = acc_v.at[par]

                def accrow(j, cc):
                    row = F * j
                    for g in range(NG):
                        sl = pl.ds(g * 16, 16)
                        l0, h0 = halves(rb[row, sl])
                        l1, h1 = halves(rb[row + 1, sl])
                        l2, h2 = halves(rb[row + 2, sl])
                        l3, h3 = halves(rb[row + 3, sl])
                        ab[j, pl.ds(g * 16, 16)] = (l0 + l1) + (l2 + l3)
                        ab[j, pl.ds(DW + g * 16, 16)] = (h0 + h1) + (h2 + h3)
                    return cc

                lax.fori_loop(0, CH, accrow, 0)
                out_desc(c, par).start()
            return carry

        lax.fori_loop(0, CHUNKS // 2, pairbody, 0)
        out_desc(CHUNKS - 2, 0).wait()
        out_desc(CHUNKS - 1, 1).wait()

    return k(t2r, ids3)


def kernel(table, conv_w, conv_b, input_ids):
    V, D = table.shape
    F = conv_w.shape[2]
    B, S = input_ids.shape
    R = B * S // F

    info = plsc.get_sparse_core_info()
    NC, NS = info.num_cores, info.num_subcores
    NW = NC * NS

    wcat = conv_w.transpose(1, 2, 0).reshape(D, F * D)
    t2r = _t2_matmul(table, wcat, conv_b.reshape(1, D), V, D, F)

    rows_w = R // NW
    CH = 8
    ids3 = input_ids.reshape(NW, rows_w // CH, CH * F).astype(jnp.int32)
    out_flat = _sc_gather_sum(t2r, ids3, R, V, D, F, NC, NS)
    return out_flat.reshape(B, S // F, D)
